# baseline (speedup 1.0000x reference)
import jax
import jax.numpy as jnp
from jax.experimental import pallas as pl
from jax.experimental.pallas import tpu as pltpu

EPS = 1e-5
_G = 4
_S = 5


def _tree_max(parts):
    while len(parts) > 1:
        odd = parts[len(parts) - len(parts) % 2:]
        parts = [jnp.maximum(parts[i], parts[i + 1])
                 for i in range(0, len(parts) - 1, 2)] + odd
    return parts[0]


def _encoder_kernel(x0_ref, x1_ref, x2_ref, x3_ref, x4_ref, m_ref,
                    w0_ref, b0_ref, w1a_ref, w1b_ref, b1_ref,
                    w2_ref, b2_ref, w3_ref, b3_ref, w4_ref, b4_ref,
                    out_ref, f_scr, g_scr):
    x_refs = (x0_ref, x1_ref, x2_ref, x3_ref, x4_ref)
    blk, CK = x_refs[0].shape
    S = _S
    H = w1a_ref.shape[1]
    P = _G * S
    m2d = m_ref[...].astype(jnp.bfloat16)
    b0 = b0_ref[...]
    for pg in range(S):
        xg = x_refs[pg][...]
        for j in range(_G):
            fj = jnp.dot(xg, w0_ref[pl.ds(j * CK, CK), :],
                         preferred_element_type=jnp.float32)
            s_idx = j * S + pg
            f_scr[pl.ds(s_idx * blk, blk), :] = (
                jnp.maximum((fj + b0).astype(jnp.bfloat16), 0))
    g_scr[...] = jnp.dot(f_scr[...], w1a_ref[...],
                         preferred_element_type=jnp.float32).astype(jnp.bfloat16)
    pcol = lambda s: 4 * (s % _S) + s // _S
    pooled = _tree_max([f_scr[pl.ds(s * blk, blk), :]
                        * m2d[:, pcol(s):pcol(s) + 1] for s in range(P)])
    pw = jnp.dot(pooled, w1b_ref[...], preferred_element_type=jnp.float32)
    pwb = (pw + b1_ref[...]).astype(jnp.bfloat16)
    g3 = g_scr[...].reshape(P, blk, H)
    h = jnp.maximum(g3 + pwb[None, :, :], 0)
    h2 = jnp.dot(h.reshape(P * blk, H), w2_ref[...],
                 preferred_element_type=jnp.float32)
    h2b = jnp.maximum(h2.astype(jnp.bfloat16) + b2_ref[...].astype(jnp.bfloat16), 0)
    h3 = h2b.reshape(P, blk, H)
    buf = _tree_max([h3[s] * m2d[:, pcol(s):pcol(s) + 1] for s in range(P)])
    o = jnp.dot(buf.astype(jnp.float32), w3_ref[...],
                preferred_element_type=jnp.float32)
    o = jnp.maximum(o + b3_ref[...], 0.0)
    o = jnp.dot(o, w4_ref[...], preferred_element_type=jnp.float32)
    o = o + b4_ref[...]
    valid = jnp.max(m_ref[...], axis=1, keepdims=True)
    out_ref[...] = o * valid


def kernel(polylines, polylines_mask, W0, g0, b0, W1, g1, b1, W2, g2, b2,
           W3, b3, W4, b4):
    N, P, C = polylines.shape
    H = W0.shape[1]
    O = W4.shape[1]
    s = 1.0 / jnp.sqrt(jnp.float32(1.0) + EPS)
    W0s = W0 * (g0 * s)[None, :]
    W1s = W1 * (g1 * s)[None, :]
    W1a, W1b = W1s[:H], W1s[H:]
    W2s = W2 * (g2 * s)[None, :]

    CK = _G * C
    w0_stack = jnp.zeros((_G, CK, H), jnp.float32)
    for j in range(_G):
        w0_stack = w0_stack.at[j, j * C:(j + 1) * C, :].set(W0s)
    w0_stack = w0_stack.reshape(_G * CK, H)

    mf = polylines_mask.astype(jnp.float32)
    xg = polylines.reshape(N, _S, CK)
    x_slices = [xg[:, pg, :].astype(jnp.bfloat16) for pg in range(_S)]

    blk = 512
    grid = (N // blk,)
    full = lambda shape: pl.BlockSpec(shape, lambda i: (0,) * len(shape))

    return pl.pallas_call(
        _encoder_kernel,
        grid=grid,
        in_specs=[
            *[pl.BlockSpec((blk, CK), lambda i: (i, 0)) for _ in range(_S)],
            pl.BlockSpec((blk, P), lambda i: (i, 0)),
            full((_G * CK, H)),
            full((1, H)),
            full((H, H)),
            full((H, H)),
            full((1, H)),
            full((H, H)),
            full((1, H)),
            full((H, H)),
            full((1, H)),
            full((H, O)),
            full((1, O)),
        ],
        out_specs=pl.BlockSpec((blk, O), lambda i: (i, 0)),
        out_shape=jax.ShapeDtypeStruct((N, O), jnp.float32),
        scratch_shapes=[pltpu.VMEM((P * blk, H), jnp.bfloat16),
                        pltpu.VMEM((P * blk, H), jnp.bfloat16)],
        compiler_params=pltpu.CompilerParams(
            dimension_semantics=("parallel",),
        ),
    )(*x_slices, mf, w0_stack.astype(jnp.bfloat16), b0.reshape(1, H),
      W1a.astype(jnp.bfloat16), W1b.astype(jnp.bfloat16), b1.reshape(1, H),
      W2s.astype(jnp.bfloat16), b2.reshape(1, H),
      W3, b3.reshape(1, H), W4, b4.reshape(1, O))


# speedup vs baseline: 1.1237x; 1.1237x over previous
import jax
import jax.numpy as jnp
from jax.experimental import pallas as pl
from jax.experimental.pallas import tpu as pltpu

EPS = 1e-5
_G = 4
_S = 5


def _tree_max(parts):
    while len(parts) > 1:
        odd = parts[len(parts) - len(parts) % 2:]
        parts = [jnp.maximum(parts[i], parts[i + 1])
                 for i in range(0, len(parts) - 1, 2)] + odd
    return parts[0]


def _encoder_kernel(x_ref, m_ref, w0_ref, b0_ref, w1a_ref, w1b_ref, b1_ref,
                    w2_ref, b2_ref, w3_ref, b3_ref, w4_ref, b4_ref,
                    out_ref, f_scr, g_scr):
    S, blk, CK = x_ref.shape
    H = w1a_ref.shape[1]
    P = _G * S
    m2d = m_ref[...].astype(jnp.bfloat16)
    x2 = x_ref[...].reshape(S * blk, CK).astype(jnp.bfloat16)
    b0 = b0_ref[...]
    for j in range(_G):
        fj = jnp.dot(x2, w0_ref[pl.ds(j * CK, CK), :],
                     preferred_element_type=jnp.float32)
        f_scr[pl.ds(j * S * blk, S * blk), :] = (
            jnp.maximum((fj + b0).astype(jnp.bfloat16), 0))
    g_scr[...] = jnp.dot(f_scr[...], w1a_ref[...],
                         preferred_element_type=jnp.float32).astype(jnp.bfloat16)
    pcol = lambda s: 4 * (s % _S) + s // _S
    pooled = _tree_max([f_scr[pl.ds(s * blk, blk), :]
                        * m2d[:, pcol(s):pcol(s) + 1] for s in range(P)])
    pw = jnp.dot(pooled, w1b_ref[...], preferred_element_type=jnp.float32)
    pwb = (pw + b1_ref[...]).astype(jnp.bfloat16)
    g3 = g_scr[...].reshape(P, blk, H)
    h = jnp.maximum(g3 + pwb[None, :, :], 0)
    h2 = jnp.dot(h.reshape(P * blk, H), w2_ref[...],
                 preferred_element_type=jnp.float32)
    h2b = jnp.maximum(h2.astype(jnp.bfloat16) + b2_ref[...].astype(jnp.bfloat16), 0)
    h3 = h2b.reshape(P, blk, H)
    buf = _tree_max([h3[s] * m2d[:, pcol(s):pcol(s) + 1] for s in range(P)])
    o = jnp.dot(buf.astype(jnp.float32), w3_ref[...],
                preferred_element_type=jnp.float32)
    o = jnp.maximum(o + b3_ref[...], 0.0)
    o = jnp.dot(o, w4_ref[...], preferred_element_type=jnp.float32)
    o = o + b4_ref[...]
    valid = jnp.max(m_ref[...], axis=1, keepdims=True)
    out_ref[...] = o * valid


def kernel(polylines, polylines_mask, W0, g0, b0, W1, g1, b1, W2, g2, b2,
           W3, b3, W4, b4):
    N, P, C = polylines.shape
    H = W0.shape[1]
    O = W4.shape[1]
    s = 1.0 / jnp.sqrt(jnp.float32(1.0) + EPS)
    W0s = W0 * (g0 * s)[None, :]
    W1s = W1 * (g1 * s)[None, :]
    W1a, W1b = W1s[:H], W1s[H:]
    W2s = W2 * (g2 * s)[None, :]

    CK = _G * C
    w0_stack = jnp.zeros((_G, CK, H), jnp.float32)
    for j in range(_G):
        w0_stack = w0_stack.at[j, j * C:(j + 1) * C, :].set(W0s)
    w0_stack = w0_stack.reshape(_G * CK, H)

    mf = polylines_mask.astype(jnp.float32)
    xq = polylines.reshape(N, _S, CK).transpose(1, 0, 2)

    blk = 512
    grid = (N // blk,)
    full = lambda shape: pl.BlockSpec(shape, lambda i: (0,) * len(shape))

    return pl.pallas_call(
        _encoder_kernel,
        grid=grid,
        in_specs=[
            pl.BlockSpec((_S, blk, CK), lambda i: (0, i, 0)),
            pl.BlockSpec((blk, P), lambda i: (i, 0)),
            full((_G * CK, H)),
            full((1, H)),
            full((H, H)),
            full((H, H)),
            full((1, H)),
            full((H, H)),
            full((1, H)),
            full((H, H)),
            full((1, H)),
            full((H, O)),
            full((1, O)),
        ],
        out_specs=pl.BlockSpec((blk, O), lambda i: (i, 0)),
        out_shape=jax.ShapeDtypeStruct((N, O), jnp.float32),
        scratch_shapes=[pltpu.VMEM((P * blk, H), jnp.bfloat16),
                        pltpu.VMEM((P * blk, H), jnp.bfloat16)],
        compiler_params=pltpu.CompilerParams(
            dimension_semantics=("parallel",),
        ),
    )(xq, mf, w0_stack.astype(jnp.bfloat16), b0.reshape(1, H),
      W1a.astype(jnp.bfloat16), W1b.astype(jnp.bfloat16), b1.reshape(1, H),
      W2s.astype(jnp.bfloat16), b2.reshape(1, H),
      W3, b3.reshape(1, H), W4, b4.reshape(1, O))
